# baseline (speedup 1.0000x reference)
import functools

import jax
import jax.numpy as jnp
from jax import lax
from jax.experimental import pallas as pl
from jax.experimental.pallas import tpu as pltpu
from jax.experimental.pallas import tpu_sc as plsc

BATCH = 16384
NG = 10
NJ = 5
W = 32

_INFO = plsc.get_sparse_core_info()
_NC, _NS = _INFO.num_cores, _INFO.num_subcores
_NW = _NC * _NS
_ROWS = BATCH // _NW


def _body(in_hbm, *out_hbms):
    wid = lax.axis_index("s") * _NC + lax.axis_index("c")
    base = wid * _ROWS
    for g in range(NG):
        for j in range(NJ):
            src_col = (j * NG + g) * W
            pltpu.sync_copy(
                in_hbm.at[pl.ds(base, _ROWS), pl.ds(src_col, W)],
                out_hbms[g].at[pl.ds(base, _ROWS), pl.ds(j * W, W)],
            )


@jax.jit
def kernel(input_tensor):
    mesh = plsc.VectorSubcoreMesh(core_axis_name="c", subcore_axis_name="s")
    out_type = tuple(
        jax.ShapeDtypeStruct((BATCH, NJ * W), jnp.float32) for _ in range(NG)
    )
    return pl.kernel(
        _body,
        out_type=out_type,
        mesh=mesh,
        compiler_params=pltpu.CompilerParams(use_tc_tiling_on_sc=False),
    )(input_tensor)
